# Initial kernel scaffold; baseline (speedup 1.0000x reference)
import jax
import jax.numpy as jnp
from jax.experimental import pallas as pl


def kernel(state, hx, cx, target_appear, target_info, target_indicator, action_probs, states_rep, states_memory, actions_memory, params):
    raise NotImplementedError("write your pallas kernel here")


# speedup vs baseline: 1.2382x; 1.2382x over previous
import jax
import jax.numpy as jnp
from jax.experimental import pallas as pl
from jax.experimental.pallas import tpu as pltpu

_NUM_CATE = 22
_HID = 512
_HEADS = 4
_K = 10
_M = 100000
_BLK = 2000
_NBLK = _M // _BLK


def _ln(x, g, b):
    m = jnp.mean(x, axis=-1, keepdims=True)
    v = jnp.var(x, axis=-1, keepdims=True)
    return (x - m) / jnp.sqrt(v + 1e-5) * g + b


def _mem_kernel(csr_ref, rep_ref, sm_hbm, am_hbm,
                att_sm_ref, att_am_ref,
                scores_ref, asm_ref, smk_ref, amk_ref, sem_s, sem_a):
    h = pl.program_id(0)
    b = pl.program_id(1)

    @pl.when(jnp.logical_and(h == 0, b == 0))
    def _init():
        asm_ref[...] = csr_ref[...]

    scores = jax.lax.dot_general(
        asm_ref[...], rep_ref[...],
        dimension_numbers=(((1,), (1,)), ((), ())),
        preferred_element_type=jnp.float32)
    scores_ref[pl.ds(b, 1), :] = scores

    @pl.when(b == _NBLK - 1)
    def _combine():
        amk_ref[...] = jnp.zeros_like(amk_ref)
        smk_ref[pl.ds(_K, 16 - _K), :] = jnp.zeros((16 - _K, _HID), jnp.float32)
        s = scores_ref[...]
        rows = jax.lax.broadcasted_iota(jnp.int32, (_NBLK, _BLK), 0)
        cols = jax.lax.broadcasted_iota(jnp.int32, (_NBLK, _BLK), 1)
        lin = rows * _BLK + cols
        vals = []
        copies = []
        for k in range(_K):
            m = jnp.max(s)
            idx = jnp.min(jnp.where(s == m, lin, jnp.int32(2 ** 30)))
            vals.append(m)
            s = jnp.where(lin == idx, -jnp.inf, s)
            cs = pltpu.make_async_copy(sm_hbm.at[pl.ds(idx, 1), :],
                                       smk_ref.at[pl.ds(k, 1), :],
                                       sem_s.at[k])
            ca = pltpu.make_async_copy(am_hbm.at[pl.ds(idx, 1), :],
                                       amk_ref.at[pl.ds(k, 1), :],
                                       sem_a.at[k])
            cs.start()
            ca.start()
            copies.append((cs, ca))
        v = jnp.concatenate([x.reshape(1, 1) for x in vals], axis=1)
        e = jnp.exp(v - vals[0])
        w10 = e / jnp.sum(e)
        w = jnp.concatenate([w10, jnp.zeros((1, 6), jnp.float32)], axis=1)
        for cs, ca in copies:
            cs.wait()
            ca.wait()
        asm_new = jax.lax.dot_general(
            w, smk_ref[...], dimension_numbers=(((1,), (0,)), ((), ())),
            preferred_element_type=jnp.float32)
        aam = jax.lax.dot_general(
            w, amk_ref[...], dimension_numbers=(((1,), (0,)), ((), ())),
            preferred_element_type=jnp.float32)
        asm_ref[...] = asm_new

        @pl.when(h == 0)
        def _first():
            att_sm_ref[...] = asm_new
            att_am_ref[...] = aam

        @pl.when(h > 0)
        def _acc():
            att_sm_ref[...] = att_sm_ref[...] + asm_new
            att_am_ref[...] = att_am_ref[...] + aam


def _memory_attention(csr, states_rep, states_memory, actions_memory,
                      interpret=False):
    am_padded = jnp.pad(actions_memory, ((0, 0), (0, 128 - 6)))
    att_sm, att_am = pl.pallas_call(
        _mem_kernel,
        grid=(_HEADS, _NBLK),
        in_specs=[
            pl.BlockSpec((1, _HID), lambda h, b: (0, 0)),
            pl.BlockSpec((_BLK, _HID), lambda h, b: (b, 0)),
            pl.BlockSpec(memory_space=pltpu.MemorySpace.HBM),
            pl.BlockSpec(memory_space=pltpu.MemorySpace.HBM),
        ],
        out_specs=[
            pl.BlockSpec((1, _HID), lambda h, b: (0, 0)),
            pl.BlockSpec((1, 128), lambda h, b: (0, 0)),
        ],
        out_shape=[
            jax.ShapeDtypeStruct((1, _HID), jnp.float32),
            jax.ShapeDtypeStruct((1, 128), jnp.float32),
        ],
        scratch_shapes=[
            pltpu.VMEM((_NBLK, _BLK), jnp.float32),
            pltpu.VMEM((1, _HID), jnp.float32),
            pltpu.VMEM((16, _HID), jnp.float32),
            pltpu.VMEM((16, 128), jnp.float32),
            pltpu.SemaphoreType.DMA((_K,)),
            pltpu.SemaphoreType.DMA((_K,)),
        ],
        compiler_params=pltpu.CompilerParams(
            dimension_semantics=("arbitrary", "arbitrary")),
        interpret=interpret,
    )(csr, states_rep, states_memory, am_padded)
    return jnp.concatenate([att_sm, att_am[:, :6]], axis=1)


def kernel(state, hx, cx, target_appear, target_info, target_indicator,
           action_probs, states_rep, states_memory, actions_memory, params):
    p = params
    relu = jax.nn.relu
    ti = jnp.concatenate([target_info, target_indicator], axis=1)
    t = relu(ti @ p['gdo1_w'].T + p['gdo1_b'])
    t = t.T
    t = relu(t @ p['gdo2_w'].T + p['gdo2_b'])
    t = relu(t @ p['gdo3_w'].T + p['gdo3_b'])
    t = relu(t @ p['gdo4_w'].T + p['gdo4_b'])
    t = relu(t @ p['gdo5_w'].T + p['gdo5_b'])
    ta = (target_appear.T @ t).T
    tgt = jnp.concatenate([ta, target_info, target_indicator], axis=1)
    tgt = relu(tgt @ p['gdf1_w'].T + p['gdf1_b'])
    tgt = relu(tgt @ p['gdf2_w'].T + p['gdf2_b'])
    target_embedding = tgt.reshape(1, _NUM_CATE, 7, 7)
    ae = relu(action_probs @ p['ea_w'].T + p['ea_b'])
    ar = jnp.tile(ae.reshape(1, 10, 1, 1), (1, 1, 7, 7))
    img = relu(jnp.einsum('bchw,oc->bohw', state, p['conv1_w'])
               + p['conv1_b'].reshape(1, -1, 1, 1))
    x = jnp.concatenate([img, target_embedding, ar], axis=1)
    x = relu(jnp.einsum('bchw,oc->bohw', x, p['pw_w'])
             + p['pw_b'].reshape(1, -1, 1, 1))
    emb = x.reshape(1, -1)

    def cell(xin, h, c, wih, whh, bih, bhh):
        g = xin @ wih.T + h @ whh.T + bih + bhh
        i_, f_, gg, o_ = jnp.split(g, 4, axis=-1)
        c2 = jax.nn.sigmoid(f_) * c + jax.nn.sigmoid(i_) * jnp.tanh(gg)
        h2 = jax.nn.sigmoid(o_) * jnp.tanh(c2)
        return h2, c2

    h0, c0 = cell(emb, hx[0], cx[0], p['wih0'], p['whh0'], p['bih0'], p['bhh0'])
    h1, c1 = cell(h0, hx[1], cx[1], p['wih1'], p['whh1'], p['bih1'], p['bhh1'])
    hx_new = jnp.stack([h0, h1])
    cx_new = jnp.stack([c0, c1])
    x512 = h1
    t1 = relu(_ln(x512 @ p['mcs1_w'].T + p['mcs1_b'],
                  p['mcs_ln1_g'], p['mcs_ln1_b']))
    t2 = _ln(t1 @ p['mcs2_w'].T + p['mcs2_b'], p['mcs_ln2_g'], p['mcs_ln2_b'])
    csr = relu(x512 + t2)

    att = _memory_attention(csr, states_rep, states_memory, actions_memory)

    h_ = relu(_ln(att @ p['mme1_w'].T + p['mme1_b'],
                  p['mme_ln1_g'], p['mme_ln1_b']))
    att2 = relu(_ln(h_ @ p['mme2_w'].T + p['mme2_b'],
                    p['mme_ln2_g'], p['mme_ln2_b']))
    msr = jnp.concatenate([csr, att2], axis=1)
    r_ = relu(_ln(msr @ p['mlr1_w'].T + p['mlr1_b'],
                  p['mlr_ln1_g'], p['mlr_ln1_b']))
    r_ = _ln(r_ @ p['mlr2_w'].T + p['mlr2_b'], p['mlr_ln2_g'], p['mlr_ln2_b'])
    msr = relu(msr + r_)
    meta_action = relu(msr @ p['mlp1_w'].T + p['mlp1_b']) @ p['mlp2_w'].T \
        + p['mlp2_b']
    actor = x512 @ p['actor_w'].T + p['actor_b']
    critic = (x512 @ p['cr1_w'].T + p['cr1_b']) @ p['cr2_w'].T + p['cr2_b']
    return actor, critic, hx_new, cx_new, img, csr, meta_action
